# baseline (speedup 1.0000x reference)
import jax
import jax.numpy as jnp
from jax.experimental import pallas as pl
from jax.experimental.pallas import tpu as pltpu


def _scan_kernel(x_ref, b_ref, g1_ref, g2_ref, g3_ref, o_ref):
    W, bcb, H = x_ref.shape

    def step(w, prev):
        x = x_ref[w]
        b = b_ref[w]
        g1 = g1_ref[w]
        g2 = g2_ref[w]
        g3 = g3_ref[w]
        zero = jnp.zeros((bcb, 1), jnp.float32)
        up = jnp.concatenate([zero, prev[:, :-1]], axis=1)
        dn = jnp.concatenate([prev[:, 1:], zero], axis=1)
        h = b * x + g1 * up + g2 * prev + g3 * dn
        o_ref[w] = h
        return h

    jax.lax.fori_loop(0, W, step, jnp.zeros((bcb, H), jnp.float32))


def kernel(X, B, G1, G2, G3):
    Bsz, C, H, W = X.shape
    BC = Bsz * C
    bcb = min(8, BC)

    def prep(t):
        return t.reshape(BC, H, W).transpose(2, 0, 1)

    ins = [prep(t) for t in (X, B, G1, G2, G3)]

    spec = pl.BlockSpec((W, bcb, H), lambda i: (0, i, 0))
    out = pl.pallas_call(
        _scan_kernel,
        grid=(BC // bcb,),
        in_specs=[spec] * 5,
        out_specs=spec,
        out_shape=jax.ShapeDtypeStruct((W, BC, H), jnp.float32),
        compiler_params=pltpu.CompilerParams(
            dimension_semantics=("parallel",),
        ),
    )(*ins)
    return out.transpose(1, 2, 0).reshape(Bsz, C, H, W)


# speedup vs baseline: 2.6470x; 2.6470x over previous
import jax
import jax.numpy as jnp
from jax.experimental import pallas as pl
from jax.experimental.pallas import tpu as pltpu


def _scan_kernel(x_ref, b_ref, g1_ref, g2_ref, g3_ref, o_ref,
                 xs, bs, g1s, g2s, g3s, os):
    bcb, H, W = x_ref.shape

    xs[...] = jnp.transpose(x_ref[...], (2, 0, 1))
    bs[...] = jnp.transpose(b_ref[...], (2, 0, 1))
    g1s[...] = jnp.transpose(g1_ref[...], (2, 0, 1))
    g2s[...] = jnp.transpose(g2_ref[...], (2, 0, 1))
    g3s[...] = jnp.transpose(g3_ref[...], (2, 0, 1))

    def step(w, prev):
        x = xs[w]
        b = bs[w]
        g1 = g1s[w]
        g2 = g2s[w]
        g3 = g3s[w]
        zero = jnp.zeros((bcb, 1), jnp.float32)
        up = jnp.concatenate([zero, prev[:, :-1]], axis=1)
        dn = jnp.concatenate([prev[:, 1:], zero], axis=1)
        h = b * x + g1 * up + g2 * prev + g3 * dn
        os[w] = h
        return h

    jax.lax.fori_loop(0, W, step, jnp.zeros((bcb, H), jnp.float32))
    o_ref[...] = jnp.transpose(os[...], (1, 2, 0))


def kernel(X, B, G1, G2, G3):
    Bsz, C, H, W = X.shape
    BC = Bsz * C
    bcb = 32

    ins = [t.reshape(BC, H, W) for t in (X, B, G1, G2, G3)]

    spec = pl.BlockSpec((bcb, H, W), lambda i: (i, 0, 0))
    scratch = [pltpu.VMEM((W, bcb, H), jnp.float32) for _ in range(6)]
    out = pl.pallas_call(
        _scan_kernel,
        grid=(BC // bcb,),
        in_specs=[spec] * 5,
        out_specs=spec,
        out_shape=jax.ShapeDtypeStruct((BC, H, W), jnp.float32),
        scratch_shapes=scratch,
        compiler_params=pltpu.CompilerParams(
            dimension_semantics=("parallel",),
            vmem_limit_bytes=100 * 1024 * 1024,
        ),
    )(*ins)
    return out.reshape(Bsz, C, H, W)
